# baseline (speedup 1.0000x reference)
import jax
import jax.numpy as jnp
from jax import lax
from jax.experimental import pallas as pl
from jax.experimental.pallas import tpu as pltpu

_B, _P, _D, _H = 16, 2048, 1024, 24
_D4 = _D // 4
_CHUNK = 256
_NCH = _P // _CHUNK
_NBUF = 8
_XDIM = _D + _H * _D4
_NW = _XDIM // _D4


def _body(n_ref, ne_ref, cb_ref, ci_ref, s_ref,
          ctx_ref, w1_ref, day_ref, wp_ref, bp_ref, b1_ref, w2_ref, b2_ref,
          out_ref, abuf, asem, wbig, wsem, acc_ref, hscr, acc2_ref):
    def chunk_copy(b, i, slot):
        return pltpu.make_async_copy(
            ctx_ref.at[b, pl.ds(i * _CHUNK, _CHUNK), :], abuf.at[slot],
            asem.at[slot])

    def w1_copy(c):
        return pltpu.make_async_copy(
            w1_ref.at[:, pl.ds(c * _D4, _D4)], wbig.at[c], wsem)

    total = s_ref[0]

    def prime(j, c):
        chunk_copy(cb_ref[j], ci_ref[j], lax.rem(j, _NBUF)).start()
        return c

    lax.fori_loop(0, jnp.minimum(total, _NBUF), prime, 0)
    for c in range(_NW):
        w1_copy(c).start()

    hvec = lax.broadcasted_iota(jnp.int32, (_H, 1), 0)

    def consume(j, c):
        slot = lax.rem(j, _NBUF)
        b = cb_ref[j]
        i = ci_ref[j]
        n = n_ref[b]
        bin_size = jnp.maximum(n // _H, 1)
        start = hvec * bin_size
        end = jnp.minimum(start + bin_size, n)
        chunk_copy(b, i, slot).wait()

        @pl.when(i == 0)
        def _():
            acc_ref[...] = jnp.zeros_like(acc_ref)

        pos = lax.broadcasted_iota(jnp.int32, (_H, _CHUNK), 1) + i * _CHUNK
        m = ((pos >= start) & (pos < end)).astype(jnp.bfloat16)
        acc_ref[...] += jnp.dot(m, abuf[slot].astype(jnp.bfloat16),
                                preferred_element_type=jnp.float32)

        @pl.when(j + _NBUF < total)
        def _():
            chunk_copy(cb_ref[j + _NBUF], ci_ref[j + _NBUF], slot).start()

        @pl.when(i + 1 == ne_ref[b])
        def _():
            inv = 1.0 / jnp.maximum((end - start).astype(jnp.float32), 1.0)
            nonempty = (start < n).astype(jnp.float32)
            seg_mean = acc_ref[...] * (inv * nonempty)
            hourly = lax.dot_general(
                seg_mean, wp_ref[...], (((1,), (1,)), ((), ())),
                preferred_element_type=jnp.float32)
            hscr[pl.ds(b, 1)] = (hourly + bp_ref[...] * nonempty)[None]
        return c

    lax.fori_loop(0, total, consume, 0)

    acc2_ref[...] = jnp.zeros_like(acc2_ref)
    for c in range(_NW):
        w1_copy(c).wait()
        if c < _D // _D4:
            xc = day_ref[:, c * _D4:(c + 1) * _D4]
        else:
            xc = hscr[:, c - _D // _D4, :]
        acc2_ref[...] += lax.dot_general(
            xc, wbig[c], (((1,), (1,)), ((), ())),
            preferred_element_type=jnp.float32)

    y = acc2_ref[...] + b1_ref[...]
    y = 0.5 * y * (1.0 + lax.erf(y * 0.7071067811865476))
    out_ref[...] = lax.dot_general(
        y, w2_ref[...], (((1,), (1,)), ((), ())),
        preferred_element_type=jnp.float32) + b2_ref[...]


def kernel(day_embed, episode_ctx, n_episodes, Wp, bp, W1, b1, W2, b2):
    n = n_episodes.astype(jnp.int32)
    bin_size = jnp.maximum(n // _H, 1)
    needed = jnp.minimum(_H * bin_size, n)
    nch = jnp.maximum((needed + _CHUNK - 1) // _CHUNK, 1)
    off = jnp.concatenate([jnp.zeros((1,), jnp.int32),
                           jnp.cumsum(nch, dtype=jnp.int32)])
    total = off[_B]
    j = jnp.arange(_B * _NCH, dtype=jnp.int32)
    cb = jnp.clip(jnp.searchsorted(off, j, side='right') - 1, 0, _B - 1)
    ci = j - off[cb]
    cb = cb.astype(jnp.int32)
    ci = ci.astype(jnp.int32)

    out = pl.pallas_call(
        _body,
        grid_spec=pltpu.PrefetchScalarGridSpec(
            num_scalar_prefetch=5,
            grid=(1,),
            in_specs=[
                pl.BlockSpec(memory_space=pl.ANY),
                pl.BlockSpec(memory_space=pl.ANY),
                pl.BlockSpec((_B, _D), lambda i, *s: (0, 0)),
                pl.BlockSpec((_D4, _D), lambda i, *s: (0, 0)),
                pl.BlockSpec((1, _D4), lambda i, *s: (0, 0)),
                pl.BlockSpec((1, _D), lambda i, *s: (0, 0)),
                pl.BlockSpec((2, _D), lambda i, *s: (0, 0)),
                pl.BlockSpec((1, 2), lambda i, *s: (0, 0)),
            ],
            out_specs=pl.BlockSpec((_B, 2), lambda i, *s: (0, 0)),
            scratch_shapes=[
                pltpu.VMEM((_NBUF, _CHUNK, _D), jnp.float32),
                pltpu.SemaphoreType.DMA((_NBUF,)),
                pltpu.VMEM((_NW, _D, _D4), jnp.float32),
                pltpu.SemaphoreType.DMA,
                pltpu.VMEM((_H, _D), jnp.float32),
                pltpu.VMEM((_B, _H, _D4), jnp.float32),
                pltpu.VMEM((_B, _D), jnp.float32),
            ],
        ),
        out_shape=jax.ShapeDtypeStruct((_B, 2), jnp.float32),
    )(n, nch, cb, ci, total.reshape(1),
      episode_ctx, W1, day_embed, Wp, bp.reshape(1, _D4),
      b1.reshape(1, _D), W2, b2.reshape(1, 2))

    return out


# speedup vs baseline: 1.0039x; 1.0039x over previous
import jax
import jax.numpy as jnp
from jax import lax
from jax.experimental import pallas as pl
from jax.experimental.pallas import tpu as pltpu

_B, _P, _D, _H = 16, 2048, 1024, 24
_D4 = _D // 4
_CHUNK = 256
_NCH = _P // _CHUNK
_NBUF = 8
_XDIM = _D + _H * _D4
_NW = _XDIM // _D4


def _body(n_ref, ne_ref, cb_ref, ci_ref, s_ref,
          ctx_ref, w1_ref, day_ref, wp_ref, bp_ref, b1_ref, w2_ref, b2_ref,
          out_ref, abuf, asem, wbig, wsem, acc_ref, hscr, acc2_ref):
    def chunk_copy(b, i, slot):
        return pltpu.make_async_copy(
            ctx_ref.at[b, pl.ds(i * _CHUNK, _CHUNK), :], abuf.at[slot],
            asem.at[slot])

    def w1_copy():
        return pltpu.make_async_copy(w1_ref, wbig, wsem)

    total = s_ref[0]

    def prime(j, c):
        chunk_copy(cb_ref[j], ci_ref[j], lax.rem(j, _NBUF)).start()
        return c

    lax.fori_loop(0, jnp.minimum(total, _NBUF), prime, 0)
    w1_copy().start()

    hvec = lax.broadcasted_iota(jnp.int32, (_H, 1), 0)

    def consume(j, c):
        slot = lax.rem(j, _NBUF)
        b = cb_ref[j]
        i = ci_ref[j]
        n = n_ref[b]
        bin_size = jnp.maximum(n // _H, 1)
        start = hvec * bin_size
        end = jnp.minimum(start + bin_size, n)
        chunk_copy(b, i, slot).wait()

        @pl.when(i == 0)
        def _():
            acc_ref[...] = jnp.zeros_like(acc_ref)

        pos = lax.broadcasted_iota(jnp.int32, (_H, _CHUNK), 1) + i * _CHUNK
        m = ((pos >= start) & (pos < end)).astype(jnp.bfloat16)
        acc_ref[...] += jnp.dot(m, abuf[slot].astype(jnp.bfloat16),
                                preferred_element_type=jnp.float32)

        @pl.when(j + _NBUF < total)
        def _():
            chunk_copy(cb_ref[j + _NBUF], ci_ref[j + _NBUF], slot).start()

        @pl.when(i + 1 == ne_ref[b])
        def _():
            inv = 1.0 / jnp.maximum((end - start).astype(jnp.float32), 1.0)
            nonempty = (start < n).astype(jnp.float32)
            seg_mean = acc_ref[...] * (inv * nonempty)
            hourly = lax.dot_general(
                seg_mean, wp_ref[...], (((1,), (1,)), ((), ())),
                preferred_element_type=jnp.float32)
            hscr[pl.ds(b, 1)] = (hourly + bp_ref[...] * nonempty)[None]
        return c

    lax.fori_loop(0, total, consume, 0)

    acc2_ref[...] = jnp.zeros_like(acc2_ref)
    w1_copy().wait()
    for c in range(_NW):
        if c < _D // _D4:
            xc = day_ref[:, c * _D4:(c + 1) * _D4]
        else:
            xc = hscr[:, c - _D // _D4, :]
        acc2_ref[...] += lax.dot_general(
            xc, wbig[:, c * _D4:(c + 1) * _D4], (((1,), (1,)), ((), ())),
            preferred_element_type=jnp.float32)

    y = acc2_ref[...] + b1_ref[...]
    y = 0.5 * y * (1.0 + lax.erf(y * 0.7071067811865476))
    out_ref[...] = lax.dot_general(
        y, w2_ref[...], (((1,), (1,)), ((), ())),
        preferred_element_type=jnp.float32) + b2_ref[...]


def kernel(day_embed, episode_ctx, n_episodes, Wp, bp, W1, b1, W2, b2):
    n = n_episodes.astype(jnp.int32)
    bin_size = jnp.maximum(n // _H, 1)
    needed = jnp.minimum(_H * bin_size, n)
    nch = jnp.maximum((needed + _CHUNK - 1) // _CHUNK, 1)
    off = jnp.concatenate([jnp.zeros((1,), jnp.int32),
                           jnp.cumsum(nch, dtype=jnp.int32)])
    total = off[_B]
    j = jnp.arange(_B * _NCH, dtype=jnp.int32)
    cb = jnp.clip(jnp.searchsorted(off, j, side='right') - 1, 0, _B - 1)
    ci = j - off[cb]
    cb = cb.astype(jnp.int32)
    ci = ci.astype(jnp.int32)

    out = pl.pallas_call(
        _body,
        grid_spec=pltpu.PrefetchScalarGridSpec(
            num_scalar_prefetch=5,
            grid=(1,),
            in_specs=[
                pl.BlockSpec(memory_space=pl.ANY),
                pl.BlockSpec(memory_space=pl.ANY),
                pl.BlockSpec((_B, _D), lambda i, *s: (0, 0)),
                pl.BlockSpec((_D4, _D), lambda i, *s: (0, 0)),
                pl.BlockSpec((1, _D4), lambda i, *s: (0, 0)),
                pl.BlockSpec((1, _D), lambda i, *s: (0, 0)),
                pl.BlockSpec((2, _D), lambda i, *s: (0, 0)),
                pl.BlockSpec((1, 2), lambda i, *s: (0, 0)),
            ],
            out_specs=pl.BlockSpec((_B, 2), lambda i, *s: (0, 0)),
            scratch_shapes=[
                pltpu.VMEM((_NBUF, _CHUNK, _D), jnp.float32),
                pltpu.SemaphoreType.DMA((_NBUF,)),
                pltpu.VMEM((_D, _XDIM), jnp.float32),
                pltpu.SemaphoreType.DMA,
                pltpu.VMEM((_H, _D), jnp.float32),
                pltpu.VMEM((_B, _H, _D4), jnp.float32),
                pltpu.VMEM((_B, _D), jnp.float32),
            ],
        ),
        out_shape=jax.ShapeDtypeStruct((_B, 2), jnp.float32),
    )(n, nch, cb, ci, total.reshape(1),
      episode_ctx, W1, day_embed, Wp, bp.reshape(1, _D4),
      b1.reshape(1, _D), W2, b2.reshape(1, 2))

    return out
